# baseline (speedup 1.0000x reference)
import functools

import jax
import jax.numpy as jnp
import numpy as np
from jax.experimental import pallas as pl
from jax.experimental.pallas import tpu as pltpu

B, H, S, D = 1, 12, 4096, 64
BLK = 64
NB = S // BLK
R = 3
NSEL = 3 + 2 + R
SCALE = 1.0 / np.sqrt(D)


NQ = 8


def _attn_body(sel_ref, q_ref, k_ref, v_ref, o_ref):
    h = pl.program_id(0)
    g = pl.program_id(1)

    outs = []
    for i in range(NQ):
        n = g * NQ + i
        q = q_ref[0, pl.ds(i * BLK, BLK), :]

        qb = q.astype(jnp.bfloat16)
        s_list = []
        v_list = []
        for j in range(NSEL):
            idx = sel_ref[h, n, j]
            off = idx * BLK
            kj = k_ref[0, pl.ds(off, BLK), :].astype(jnp.bfloat16)
            v_list.append(v_ref[0, pl.ds(off, BLK), :].astype(jnp.bfloat16))
            s_list.append(jax.lax.dot_general(
                qb, kj, (((1,), (1,)), ((), ())),
                preferred_element_type=jnp.float32) * SCALE)
        m = s_list[0].max(axis=-1, keepdims=True)
        for j in range(1, NSEL):
            m = jnp.maximum(m, s_list[j].max(axis=-1, keepdims=True))
        acc = None
        l = None
        for j in range(NSEL):
            p = jnp.exp(s_list[j] - m)
            lj = jnp.sum(p, axis=-1, keepdims=True)
            oj = jax.lax.dot_general(
                p.astype(jnp.bfloat16), v_list[j], (((1,), (0,)), ((), ())),
                preferred_element_type=jnp.float32)
            acc = oj if acc is None else acc + oj
            l = lj if l is None else l + lj
        outs.append(acc / l)
    o_ref[0] = jnp.concatenate(outs, axis=0)


@jax.jit
def kernel(q, k, v, rand_attn):
    qh = q.reshape(H, S, D)
    kh = k.reshape(H, S, D)
    vh = v.reshape(H, S, D)

    blk_ids = jnp.arange(NB, dtype=jnp.int32)
    win = jnp.stack([(blk_ids - 1) % NB, blk_ids, (blk_ids + 1) % NB], axis=-1)
    glob = jnp.broadcast_to(jnp.array([0, NB - 1], jnp.int32), (NB, 2))
    fixed = jnp.broadcast_to(
        jnp.concatenate([win, glob], axis=-1)[None], (H, NB, 5))
    sel = jnp.concatenate([fixed, rand_attn.astype(jnp.int32)], axis=-1)

    grid_spec = pltpu.PrefetchScalarGridSpec(
        num_scalar_prefetch=1,
        grid=(H, NB // NQ),
        in_specs=[
            pl.BlockSpec((1, NQ * BLK, D), lambda h, g, sel: (h, g, 0)),
            pl.BlockSpec((1, S, D), lambda h, g, sel: (h, 0, 0)),
            pl.BlockSpec((1, S, D), lambda h, g, sel: (h, 0, 0)),
        ],
        out_specs=pl.BlockSpec((1, NQ * BLK, D), lambda h, g, sel: (h, g, 0)),
    )
    out = pl.pallas_call(
        _attn_body,
        grid_spec=grid_spec,
        out_shape=jax.ShapeDtypeStruct((H, S, D), jnp.float32),
    )(sel, qh, kh, vh)
    return out.reshape(B, H, S, D)


# speedup vs baseline: 1.1798x; 1.1798x over previous
import functools

import jax
import jax.numpy as jnp
import numpy as np
from jax.experimental import pallas as pl
from jax.experimental.pallas import tpu as pltpu

B, H, S, D = 1, 12, 4096, 64
BLK = 64
NB = S // BLK
R = 3
NSEL = 3 + 2 + R
SCALE = 1.0 / np.sqrt(D)


NQ = 16


def _attn_body(sel_ref, q_ref, k_ref, v_ref, o_ref):
    h = pl.program_id(0)
    g = pl.program_id(1)

    outs = []
    for i in range(NQ):
        n = g * NQ + i
        q = q_ref[0, pl.ds(i * BLK, BLK), :]

        k_blocks = []
        v_blocks = []
        for j in range(NSEL):
            idx = sel_ref[h, n, j]
            off = idx * BLK
            k_blocks.append(k_ref[0, pl.ds(off, BLK), :])
            v_blocks.append(v_ref[0, pl.ds(off, BLK), :])
        ks = jnp.concatenate(k_blocks, axis=0)
        vs = jnp.concatenate(v_blocks, axis=0)

        scores = jax.lax.dot_general(
            q.astype(jnp.bfloat16), ks.astype(jnp.bfloat16),
            (((1,), (1,)), ((), ())),
            preferred_element_type=jnp.float32) * SCALE
        m = jnp.max(scores, axis=-1, keepdims=True)
        p = jnp.exp(scores - m)
        l = jnp.sum(p, axis=-1, keepdims=True)
        out = jax.lax.dot_general(
            p.astype(jnp.bfloat16), vs.astype(jnp.bfloat16),
            (((1,), (0,)), ((), ())),
            preferred_element_type=jnp.float32)
        outs.append(out / l)
    o_ref[0] = jnp.concatenate(outs, axis=0)


@jax.jit
def kernel(q, k, v, rand_attn):
    qh = q.reshape(H, S, D)
    kh = k.reshape(H, S, D)
    vh = v.reshape(H, S, D)

    blk_ids = jnp.arange(NB, dtype=jnp.int32)
    win = jnp.stack([(blk_ids - 1) % NB, blk_ids, (blk_ids + 1) % NB], axis=-1)
    glob = jnp.broadcast_to(jnp.array([0, NB - 1], jnp.int32), (NB, 2))
    fixed = jnp.broadcast_to(
        jnp.concatenate([win, glob], axis=-1)[None], (H, NB, 5))
    sel = jnp.concatenate([fixed, rand_attn.astype(jnp.int32)], axis=-1)

    grid_spec = pltpu.PrefetchScalarGridSpec(
        num_scalar_prefetch=1,
        grid=(H, NB // NQ),
        in_specs=[
            pl.BlockSpec((1, NQ * BLK, D), lambda h, g, sel: (h, g, 0)),
            pl.BlockSpec((1, S, D), lambda h, g, sel: (h, 0, 0)),
            pl.BlockSpec((1, S, D), lambda h, g, sel: (h, 0, 0)),
        ],
        out_specs=pl.BlockSpec((1, NQ * BLK, D), lambda h, g, sel: (h, g, 0)),
    )
    out = pl.pallas_call(
        _attn_body,
        grid_spec=grid_spec,
        out_shape=jax.ShapeDtypeStruct((H, S, D), jnp.float32),
    )(sel, qh, kh, vh)
    return out.reshape(B, H, S, D)


# speedup vs baseline: 2.1585x; 1.8296x over previous
import functools

import jax
import jax.numpy as jnp
import numpy as np
from jax.experimental import pallas as pl
from jax.experimental.pallas import tpu as pltpu

B, H, S, D = 1, 12, 4096, 64
BLK = 64
NB = S // BLK
R = 3
NSEL = 3 + 2 + R
SCALE = 1.0 / np.sqrt(D)


NQ = 16


def _attn_body(sel_ref, q_ref, k_ref, v_ref, o_ref, kb_ref, vb_ref):
    h = pl.program_id(0)
    g = pl.program_id(1)

    @pl.when(g == 0)
    def _stage():
        kb_ref[...] = k_ref[0].astype(jnp.bfloat16)
        vb_ref[...] = v_ref[0].astype(jnp.bfloat16)

    outs = []
    for i in range(NQ):
        n = g * NQ + i
        q = q_ref[0, pl.ds(i * BLK, BLK), :]
        qb = (q * SCALE).astype(jnp.bfloat16)

        k_blocks = []
        v_blocks = []
        for j in range(NSEL):
            idx = sel_ref[h, n, j]
            off = idx * BLK
            k_blocks.append(kb_ref[pl.ds(off, BLK), :])
            v_blocks.append(vb_ref[pl.ds(off, BLK), :])
        ks = jnp.concatenate(k_blocks, axis=0)
        vs = jnp.concatenate(v_blocks, axis=0)

        scores = jax.lax.dot_general(
            qb, ks, (((1,), (1,)), ((), ())),
            preferred_element_type=jnp.float32)
        m = jnp.max(scores, axis=-1, keepdims=True)
        p = jnp.exp(scores - m)
        l = jnp.sum(p, axis=-1, keepdims=True)
        out = jax.lax.dot_general(
            p.astype(jnp.bfloat16), vs, (((1,), (0,)), ((), ())),
            preferred_element_type=jnp.float32)
        outs.append(out / l)
    o_ref[0] = jnp.concatenate(outs, axis=0)


@jax.jit
def kernel(q, k, v, rand_attn):
    qh = q.reshape(H, S, D)
    kh = k.reshape(H, S, D)
    vh = v.reshape(H, S, D)

    blk_ids = jnp.arange(NB, dtype=jnp.int32)
    win = jnp.stack([(blk_ids - 1) % NB, blk_ids, (blk_ids + 1) % NB], axis=-1)
    glob = jnp.broadcast_to(jnp.array([0, NB - 1], jnp.int32), (NB, 2))
    fixed = jnp.broadcast_to(
        jnp.concatenate([win, glob], axis=-1)[None], (H, NB, 5))
    sel = jnp.concatenate([fixed, rand_attn.astype(jnp.int32)], axis=-1)

    grid_spec = pltpu.PrefetchScalarGridSpec(
        num_scalar_prefetch=1,
        grid=(H, NB // NQ),
        in_specs=[
            pl.BlockSpec((1, NQ * BLK, D), lambda h, g, sel: (h, g, 0)),
            pl.BlockSpec((1, S, D), lambda h, g, sel: (h, 0, 0)),
            pl.BlockSpec((1, S, D), lambda h, g, sel: (h, 0, 0)),
        ],
        out_specs=pl.BlockSpec((1, NQ * BLK, D), lambda h, g, sel: (h, g, 0)),
        scratch_shapes=[
            pltpu.VMEM((S, D), jnp.bfloat16),
            pltpu.VMEM((S, D), jnp.bfloat16),
        ],
    )
    out = pl.pallas_call(
        _attn_body,
        grid_spec=grid_spec,
        out_shape=jax.ShapeDtypeStruct((H, S, D), jnp.float32),
    )(sel, qh, kh, vh)
    return out.reshape(B, H, S, D)


# speedup vs baseline: 2.1592x; 1.0003x over previous
import functools

import jax
import jax.numpy as jnp
import numpy as np
from jax.experimental import pallas as pl
from jax.experimental.pallas import tpu as pltpu

B, H, S, D = 1, 12, 4096, 64
BLK = 64
NB = S // BLK
R = 3
NSEL = 3 + 2 + R
SCALE = 1.0 / np.sqrt(D)

NQ = 16


def _attn_body(sel_ref, q_ref, k_ref, v_ref, o_ref):
    h = pl.program_id(0)
    g = pl.program_id(1)

    ks_list = []
    vs_list = []
    for i in range(NQ):
        n = g * NQ + i
        k_blocks = []
        v_blocks = []
        for j in range(NSEL):
            off = sel_ref[h, n, j] * BLK
            k_blocks.append(k_ref[0, pl.ds(off, BLK), :])
            v_blocks.append(v_ref[0, pl.ds(off, BLK), :])
        ks_list.append(jnp.concatenate(k_blocks, axis=0))
        vs_list.append(jnp.concatenate(v_blocks, axis=0))

    scores_list = []
    for i in range(NQ):
        qb = q_ref[0, pl.ds(i * BLK, BLK), :]
        scores_list.append(jax.lax.dot_general(
            qb, ks_list[i], (((1,), (1,)), ((), ())),
            preferred_element_type=jnp.float32))

    p_list = []
    l_list = []
    for i in range(NQ):
        s = scores_list[i]
        m = jnp.max(s, axis=-1, keepdims=True)
        p = jnp.exp(s - m)
        l_list.append(jnp.sum(p, axis=-1, keepdims=True))
        p_list.append(p.astype(jnp.bfloat16))

    for i in range(NQ):
        out = jax.lax.dot_general(
            p_list[i], vs_list[i], (((1,), (0,)), ((), ())),
            preferred_element_type=jnp.float32)
        o_ref[0, pl.ds(i * BLK, BLK), :] = out / l_list[i]


@jax.jit
def kernel(q, k, v, rand_attn):
    qh = (q.reshape(H, S, D) * SCALE).astype(jnp.bfloat16)
    kh = k.reshape(H, S, D).astype(jnp.bfloat16)
    vh = v.reshape(H, S, D).astype(jnp.bfloat16)

    blk_ids = jnp.arange(NB, dtype=jnp.int32)
    win = jnp.stack([(blk_ids - 1) % NB, blk_ids, (blk_ids + 1) % NB], axis=-1)
    glob = jnp.broadcast_to(jnp.array([0, NB - 1], jnp.int32), (NB, 2))
    fixed = jnp.broadcast_to(
        jnp.concatenate([win, glob], axis=-1)[None], (H, NB, 5))
    sel = jnp.concatenate([fixed, rand_attn.astype(jnp.int32)], axis=-1)

    grid_spec = pltpu.PrefetchScalarGridSpec(
        num_scalar_prefetch=1,
        grid=(H, NB // NQ),
        in_specs=[
            pl.BlockSpec((1, NQ * BLK, D), lambda h, g, sel: (h, g, 0)),
            pl.BlockSpec((1, S, D), lambda h, g, sel: (h, 0, 0)),
            pl.BlockSpec((1, S, D), lambda h, g, sel: (h, 0, 0)),
        ],
        out_specs=pl.BlockSpec((1, NQ * BLK, D), lambda h, g, sel: (h, g, 0)),
    )
    out = pl.pallas_call(
        _attn_body,
        grid_spec=grid_spec,
        out_shape=jax.ShapeDtypeStruct((H, S, D), jnp.float32),
    )(sel, qh, kh, vh)
    return out.reshape(B, H, S, D)
